# baseline (device time: 16159 ns/iter reference)
import jax
import jax.numpy as jnp
from jax import lax
from jax.experimental import pallas as pl
from jax.experimental.pallas import tpu as pltpu

B, SQ, H, D = 8, 1, 8, 64
SK = 512
SCALE = D ** -0.5


def kernel(Q, K, V):
    Qh = Q.reshape(B, H, D)
    Kt = K.transpose(0, 2, 3, 1)
    Vt = V.transpose(0, 2, 3, 1)

    def body(q_ref, k_ref, v_ref, out_ref, send_buf, recv_buf, send_sems, recv_sems):
        i = pl.program_id(0)
        my_x = lax.axis_index("x")
        my_y = lax.axis_index("y")
        my_z = lax.axis_index("z")
        partner = (my_x, my_y, 1 - my_z)

        @pl.when(i == 0)
        def _():
            barrier_sem = pltpu.get_barrier_semaphore()
            pl.semaphore_signal(
                barrier_sem, inc=1,
                device_id=partner,
                device_id_type=pl.DeviceIdType.MESH,
            )
            pl.semaphore_wait(barrier_sem, 1)

        q4 = q_ref[pl.ds(i, 1), :, :][:, :, :, None]
        s = jnp.sum(q4 * k_ref[...], axis=2) * SCALE
        m = jnp.max(s, axis=-1, keepdims=True)
        p = jnp.exp(s - m)
        l = jnp.sum(p, axis=-1, keepdims=True)
        o = jnp.sum(p[:, :, None, :] * v_ref[...], axis=-1)

        send_buf[pl.ds(i, 1), :, 0:D] = o
        send_buf[pl.ds(i, 1), :, D:D + 1] = m
        send_buf[pl.ds(i, 1), :, D + 1:D + 2] = l

        rdma = pltpu.make_async_remote_copy(
            src_ref=send_buf.at[i],
            dst_ref=recv_buf.at[i],
            send_sem=send_sems.at[i],
            recv_sem=recv_sems.at[i],
            device_id=partner,
            device_id_type=pl.DeviceIdType.MESH,
        )
        rdma.start()

        @pl.when(i == B - 1)
        def _():
            for j in range(B):
                w = pltpu.make_async_remote_copy(
                    src_ref=send_buf.at[j],
                    dst_ref=recv_buf.at[j],
                    send_sem=send_sems.at[j],
                    recv_sem=recv_sems.at[j],
                    device_id=partner,
                    device_id_type=pl.DeviceIdType.MESH,
                )
                w.wait_send()
                w.wait_recv()

            o_a = send_buf[:, :, 0:D]
            m_a = send_buf[:, :, D:D + 1]
            l_a = send_buf[:, :, D + 1:D + 2]
            o_b = recv_buf[:, :, 0:D]
            m_b = recv_buf[:, :, D:D + 1]
            l_b = recv_buf[:, :, D + 1:D + 2]
            m_n = jnp.maximum(m_a, m_b)
            alpha = jnp.exp(m_a - m_n)
            beta = jnp.exp(m_b - m_n)
            l_n = alpha * l_a + beta * l_b
            out_ref[...] = (alpha * o_a + beta * o_b) / l_n

    out = pl.pallas_call(
        body,
        grid=(B,),
        out_shape=jax.ShapeDtypeStruct((B, H, D), jnp.float32),
        in_specs=[
            pl.BlockSpec((B, H, D), lambda i: (0, 0, 0)),
            pl.BlockSpec((1, H, D, SK), lambda i: (i, 0, 0, 0)),
            pl.BlockSpec((1, H, D, SK), lambda i: (i, 0, 0, 0)),
        ],
        out_specs=pl.BlockSpec((B, H, D), lambda i: (0, 0, 0)),
        scratch_shapes=[
            pltpu.VMEM((B, H, 128), jnp.float32),
            pltpu.VMEM((B, H, 128), jnp.float32),
            pltpu.SemaphoreType.DMA((B,)),
            pltpu.SemaphoreType.DMA((B,)),
        ],
        compiler_params=pltpu.CompilerParams(
            collective_id=0,
            dimension_semantics=("arbitrary",),
        ),
    )(Qh, Kt, Vt)
    return out.reshape(B, SQ, H, D)


# device time: 14561 ns/iter; 1.1097x vs baseline; 1.1097x over previous
import jax
import jax.numpy as jnp
from jax import lax
from jax.experimental import pallas as pl
from jax.experimental.pallas import tpu as pltpu

B, SQ, H, D = 8, 1, 8, 64
SK = 512
SCALE = D ** -0.5
NC = 2
CB = B // NC


def kernel(Q, K, V):
    Qh = Q.reshape(B, H, D)
    Kt = K.transpose(0, 2, 3, 1)
    Vt = V.transpose(0, 2, 3, 1)

    def body(q_ref, k_ref, v_ref, out_ref, send_buf, recv_buf, send_sems, recv_sems):
        my_x = lax.axis_index("x")
        my_y = lax.axis_index("y")
        my_z = lax.axis_index("z")
        partner = (my_x, my_y, 1 - my_z)

        barrier_sem = pltpu.get_barrier_semaphore()
        pl.semaphore_signal(
            barrier_sem, inc=1,
            device_id=partner,
            device_id_type=pl.DeviceIdType.MESH,
        )

        def exchange(j):
            sl = pl.ds(j * CB, CB)
            return pltpu.make_async_remote_copy(
                src_ref=send_buf.at[sl],
                dst_ref=recv_buf.at[sl],
                send_sem=send_sems.at[j],
                recv_sem=recv_sems.at[j],
                device_id=partner,
                device_id_type=pl.DeviceIdType.MESH,
            )

        for c in range(NC):
            sl = pl.ds(c * CB, CB)
            q4 = q_ref[sl, :, :][:, :, :, None]
            s = jnp.sum(q4 * k_ref[sl], axis=2) * SCALE
            m = jnp.max(s, axis=-1, keepdims=True)
            p = jnp.exp(s - m)
            l = jnp.sum(p, axis=-1, keepdims=True)
            o = jnp.sum(p[:, :, None, :] * v_ref[sl], axis=-1)

            send_buf[sl, :, 0:D] = o
            send_buf[sl, :, D:D + 1] = m
            send_buf[sl, :, D + 1:D + 2] = l

            if c == 0:
                pl.semaphore_wait(barrier_sem, 1)
            exchange(c).start()

        for j in range(NC):
            w = exchange(j)
            w.wait_send()
            w.wait_recv()

        o_a = send_buf[:, :, 0:D]
        m_a = send_buf[:, :, D:D + 1]
        l_a = send_buf[:, :, D + 1:D + 2]
        o_b = recv_buf[:, :, 0:D]
        m_b = recv_buf[:, :, D:D + 1]
        l_b = recv_buf[:, :, D + 1:D + 2]
        m_n = jnp.maximum(m_a, m_b)
        alpha = jnp.exp(m_a - m_n)
        beta = jnp.exp(m_b - m_n)
        l_n = alpha * l_a + beta * l_b
        out_ref[...] = (alpha * o_a + beta * o_b) / l_n

    out = pl.pallas_call(
        body,
        out_shape=jax.ShapeDtypeStruct((B, H, D), jnp.float32),
        in_specs=[
            pl.BlockSpec(memory_space=pltpu.VMEM),
            pl.BlockSpec(memory_space=pltpu.VMEM),
            pl.BlockSpec(memory_space=pltpu.VMEM),
        ],
        out_specs=pl.BlockSpec(memory_space=pltpu.VMEM),
        scratch_shapes=[
            pltpu.VMEM((B, H, 128), jnp.float32),
            pltpu.VMEM((B, H, 128), jnp.float32),
            pltpu.SemaphoreType.DMA((NC,)),
            pltpu.SemaphoreType.DMA((NC,)),
        ],
        compiler_params=pltpu.CompilerParams(collective_id=0),
    )(Qh, Kt, Vt)
    return out.reshape(B, SQ, H, D)
